# baseline (speedup 1.0000x reference)
import functools

import jax
import jax.numpy as jnp
from jax import lax
from jax.experimental import pallas as pl
from jax.experimental.pallas import tpu as pltpu
from jax.experimental.pallas import tpu_sc as plsc

EPS = 1e-12
NC, NS, L = 2, 16, 16
NW = NC * NS
S_TC = 448
CB = 16


def _tc_ln_kernel(t_ref, x_ref, pos_ref, type_ref, g_ref, b_ref, o_ref):
    trow = jnp.where(t_ref[0] == 1, type_ref[1, :], type_ref[0, :])
    add = pos_ref[...] + trow[None, :]
    h = x_ref[...] + add[:, None, :]
    mu = jnp.mean(h, axis=-1, keepdims=True)
    d = h - mu
    var = jnp.mean(d * d, axis=-1, keepdims=True)
    o_ref[...] = d * jax.lax.rsqrt(var + EPS) * g_ref[...] + b_ref[...]


def _lane_allreduce_sum(v):
    dnums = lax.GatherDimensionNumbers(
        offset_dims=(), collapsed_slice_dims=(0,), start_index_map=(0,))
    for k in range(4):
        idx = jnp.arange(L, dtype=jnp.int32) ^ (1 << k)
        v = v + lax.gather(v, idx[:, None], dnums, slice_sizes=(1,),
                           mode=lax.GatherScatterMode.PROMISE_IN_BOUNDS)
    return v


def _sc_ln_body(tf_hbm, x_hbm, pos_hbm, tt_hbm, g_hbm, b_hbm, out_hbm,
                xbuf, obuf, arow, grow, brow, ttbuf, tfbuf):
    F = out_hbm.shape[0]
    B = x_hbm.shape[1]
    D = x_hbm.shape[2]
    s_tc = x_hbm.shape[0] - F
    spw = F // NW
    nj = D // L
    wid = lax.axis_index("s") * NC + lax.axis_index("c")

    pltpu.sync_copy(tt_hbm, ttbuf)
    pltpu.sync_copy(g_hbm, grow)
    pltpu.sync_copy(b_hbm, brow)
    pltpu.sync_copy(tf_hbm, tfbuf)
    tf = tfbuf[...]

    def seq_body(si, _):
        s = wid * spw + si

        def jrow(j, _):
            t0 = ttbuf[0, pl.ds(j * L, L)]
            t1 = ttbuf[1, pl.ds(j * L, L)]
            arow[pl.ds(j * L, L)] += jnp.where(tf == 1, t1, t0)
            return 0

        pltpu.sync_copy(pos_hbm.at[s_tc + s], arow)
        lax.fori_loop(0, nj, jrow, 0)

        def chunk_body(ci, _):
            b0 = ci * CB
            pltpu.sync_copy(x_hbm.at[s_tc + s, pl.ds(b0, CB)], xbuf)

            def row_body(r, _):
                def p1(j, carry):
                    acc, acc2 = carry
                    v = xbuf[r, pl.ds(j * L, L)] + arow[pl.ds(j * L, L)]
                    return acc + v, acc2 + v * v

                zero = jnp.zeros((L,), jnp.float32)
                acc, acc2 = lax.fori_loop(0, nj, p1, (zero, zero))
                mu_v = _lane_allreduce_sum(acc) / D
                var_v = _lane_allreduce_sum(acc2) / D - mu_v * mu_v
                vv = var_v + EPS
                iy = lax.bitcast_convert_type(vv, jnp.int32)
                y = lax.bitcast_convert_type(
                    jnp.int32(0x5F3759DF) - lax.shift_right_logical(iy, 1),
                    jnp.float32)
                for _ in range(4):
                    y = y * (1.5 - 0.5 * vv * y * y)

                def p2(j, _):
                    v = xbuf[r, pl.ds(j * L, L)] + arow[pl.ds(j * L, L)]
                    o = (v - mu_v) * y
                    obuf[r, pl.ds(j * L, L)] = (o * grow[pl.ds(j * L, L)]
                                                + brow[pl.ds(j * L, L)])
                    return 0

                lax.fori_loop(0, nj, p2, 0)
                return 0

            lax.fori_loop(0, CB, row_body, 0)
            pltpu.sync_copy(obuf, out_hbm.at[s, pl.ds(b0, CB)])
            return 0

        lax.fori_loop(0, B // CB, chunk_body, 0)
        return 0

    lax.fori_loop(0, spw, seq_body, 0)


def kernel(x, token_type, pos_table, type_table, ln_gamma, ln_beta):
    S, B, D = x.shape
    F = S - S_TC
    BS = 16
    t = jnp.asarray(token_type, jnp.int32).reshape((1,))
    o1 = pl.pallas_call(
        _tc_ln_kernel,
        grid_spec=pltpu.PrefetchScalarGridSpec(
            num_scalar_prefetch=1,
            grid=(S_TC // BS,),
            in_specs=[
                pl.BlockSpec((BS, B, D), lambda i, t: (i, 0, 0)),
                pl.BlockSpec((BS, D), lambda i, t: (i, 0)),
                pl.BlockSpec((2, D), lambda i, t: (0, 0)),
                pl.BlockSpec((1, 1, D), lambda i, t: (0, 0, 0)),
                pl.BlockSpec((1, 1, D), lambda i, t: (0, 0, 0)),
            ],
            out_specs=pl.BlockSpec((BS, B, D), lambda i, t: (i, 0, 0)),
        ),
        out_shape=jax.ShapeDtypeStruct((S_TC, B, D), x.dtype),
        compiler_params=pltpu.CompilerParams(
            dimension_semantics=("parallel",),
        ),
    )(t, x, pos_table, type_table,
      ln_gamma.reshape(1, 1, D), ln_beta.reshape(1, 1, D))

    tfv = jnp.full((L,), jnp.asarray(token_type, jnp.int32))
    sc_fn = pl.kernel(
        _sc_ln_body,
        out_type=jax.ShapeDtypeStruct((F, B, D), x.dtype),
        mesh=plsc.VectorSubcoreMesh(core_axis_name="c", subcore_axis_name="s"),
        scratch_types=[
            pltpu.VMEM((CB, D), jnp.float32),
            pltpu.VMEM((CB, D), jnp.float32),
            pltpu.VMEM((D,), jnp.float32),
            pltpu.VMEM((D,), jnp.float32),
            pltpu.VMEM((D,), jnp.float32),
            pltpu.VMEM((2, D), jnp.float32),
            pltpu.VMEM((L,), jnp.int32),
        ],
    )
    o2 = sc_fn(tfv, x, pos_table, type_table, ln_gamma, ln_beta)
    return jnp.concatenate([o1, o2], axis=0)


# speedup vs baseline: 2.7230x; 2.7230x over previous
import jax
import jax.numpy as jnp
from jax.experimental import pallas as pl
from jax.experimental.pallas import tpu as pltpu

EPS = 1e-12


def _fused_ln_kernel(t_ref, x_ref, pos_ref, type_ref, g_ref, b_ref, o_ref):
    trow = jnp.where(t_ref[0] == 1, type_ref[1, :], type_ref[0, :])
    add = pos_ref[...] + trow[None, :]
    h = x_ref[...] + add[:, None, :]
    mu = jnp.mean(h, axis=-1, keepdims=True)
    d = h - mu
    var = jnp.mean(d * d, axis=-1, keepdims=True)
    o_ref[...] = d * jax.lax.rsqrt(var + EPS) * g_ref[...] + b_ref[...]


def kernel(x, token_type, pos_table, type_table, ln_gamma, ln_beta):
    S, B, D = x.shape
    BS = 16
    t = jnp.asarray(token_type, jnp.int32).reshape((1,))
    out = pl.pallas_call(
        _fused_ln_kernel,
        grid_spec=pltpu.PrefetchScalarGridSpec(
            num_scalar_prefetch=1,
            grid=(S // BS,),
            in_specs=[
                pl.BlockSpec((BS, B, D), lambda i, t: (i, 0, 0)),
                pl.BlockSpec((BS, D), lambda i, t: (i, 0)),
                pl.BlockSpec((2, D), lambda i, t: (0, 0)),
                pl.BlockSpec((1, 1, D), lambda i, t: (0, 0, 0)),
                pl.BlockSpec((1, 1, D), lambda i, t: (0, 0, 0)),
            ],
            out_specs=pl.BlockSpec((BS, B, D), lambda i, t: (i, 0, 0)),
        ),
        out_shape=jax.ShapeDtypeStruct(x.shape, x.dtype),
        compiler_params=pltpu.CompilerParams(
            dimension_semantics=("parallel",),
        ),
    )(t, x, pos_table, type_table,
      ln_gamma.reshape(1, 1, D), ln_beta.reshape(1, 1, D))
    return out
